# baseline (speedup 1.0000x reference)
import dataclasses
import functools
import math

import jax
import jax.numpy as jnp
from jax import lax
from jax.experimental import pallas as pl
from jax.experimental.pallas import tpu as pltpu
from jax.experimental.pallas import tpu_sc as plsc

NC = 2
NS = 16
NW = NC * NS
L = 16
CHUNK = 128
GRP = 8

_MESH = plsc.VectorSubcoreMesh(core_axis_name="core", subcore_axis_name="subcore")

_SC_CP = pltpu.CompilerParams()
if "needs_layout_passes" in pltpu.CompilerParams.__dataclass_fields__:
    _SC_CP = dataclasses.replace(_SC_CP, needs_layout_passes=False)


def _round_up(a, b):
    return -(-a // b) * b


@functools.lru_cache(maxsize=None)
def _make_deg(N, CH):
    NPAD = _round_up(N, NS * CHUNK)
    SEG = NPAD // NS

    @functools.partial(
        pl.kernel,
        out_type=jax.ShapeDtypeStruct((NC, NPAD, L), jnp.float32),
        mesh=_MESH,
        compiler_params=_SC_CP,
        scratch_types=[
            pltpu.VMEM((CH, CHUNK), jnp.int32),
            pltpu.VMEM((CH, CHUNK), jnp.float32),
            pltpu.VMEM((CHUNK, L), jnp.float32),
            pltpu.VMEM_SHARED((NPAD, L), jnp.float32),
        ],
    )
    def deg_kernel(colp, ewp, out, colv, ewv, buf, acc):
        core = lax.axis_index("core")
        sid = lax.axis_index("subcore")
        wid = core * NS + sid

        @pl.loop(0, CHUNK)
        def _(r):
            buf[r, :] = jnp.zeros((L,), jnp.float32)

        for k in range(SEG // CHUNK):
            pltpu.sync_copy(buf, acc.at[pl.ds(sid * SEG + k * CHUNK, CHUNK)])
        plsc.subcore_barrier()

        pltpu.sync_copy(colp.at[wid], colv)
        pltpu.sync_copy(ewp.at[wid], ewv)
        iota = lax.iota(jnp.int32, L)
        zcol = jnp.zeros((L,), jnp.int32)

        @pl.loop(0, CH)
        def _(i):
            for g in range(CHUNK // L):
                w = ewv[i, pl.ds(g * L, L)]
                plsc.store_scatter(buf, [g * L + iota, zcol], w)
            pltpu.sync_copy(buf, acc.at[colv.at[i]], add=True)

        plsc.subcore_barrier()
        pltpu.sync_copy(acc.at[pl.ds(sid * SEG, SEG)],
                        out.at[core, pl.ds(sid * SEG, SEG)])

    return deg_kernel


@functools.lru_cache(maxsize=None)
def _make_agg(N, D, CH):
    GCH = CH // GRP
    NPAD = _round_up(N, NS * CHUNK)
    NT = NPAD // NS
    sizes = [CHUNK] * (NT // CHUNK) + ([NT % CHUNK] if NT % CHUNK else [])

    @functools.partial(
        pl.kernel,
        out_type=jax.ShapeDtypeStruct((NC, NPAD, D), jnp.float32),
        mesh=_MESH,
        compiler_params=_SC_CP,
        scratch_types=[
            pltpu.VMEM((GRP, CHUNK), jnp.int32),
            pltpu.VMEM((GRP, CHUNK), jnp.int32),
            pltpu.VMEM((GRP, CHUNK), jnp.int32),
            pltpu.VMEM((GRP, CHUNK), jnp.int32),
            pltpu.VMEM((GRP, CHUNK), jnp.float32),
            pltpu.VMEM((GRP, CHUNK), jnp.float32),
            pltpu.VMEM((CHUNK, D), jnp.float32),
            pltpu.VMEM((CHUNK, D), jnp.float32),
            pltpu.VMEM((1, CHUNK), jnp.int32),
            pltpu.VMEM_SHARED((NPAD, D), jnp.float32),
            pltpu.SemaphoreType.DMA,
            pltpu.SemaphoreType.DMA,
            pltpu.SemaphoreType.DMA,
            pltpu.SemaphoreType.DMA,
            pltpu.SemaphoreType.DMA,
            pltpu.SemaphoreType.DMA,
        ],
    )
    def agg_kernel(hs, rowp, colp, ewp, out,
                   rowgA, rowgB, colgA, colgB, ewgA, ewgB,
                   rbuf0, rbuf1, colx, acc, isA, isB, gs0, gs1, cs0, cs1):
        core = lax.axis_index("core")
        sid = lax.axis_index("subcore")
        wid = core * NS + sid

        @pl.loop(0, CHUNK)
        def _(r):
            for j in range(D // L):
                rbuf0[r, pl.ds(j * L, L)] = jnp.zeros((L,), jnp.float32)

        off = 0
        for sz in sizes:
            pltpu.sync_copy(rbuf0.at[pl.ds(0, sz)],
                            acc.at[pl.ds(sid * NT + off, sz)])
            off += sz
        plsc.subcore_barrier()

        A = (rowgA, colgA, ewgA, isA)
        B = (rowgB, colgB, ewgB, isB)

        def fetch_group(g, bufs):
            rowg, colg, ewg, sem = bufs
            pltpu.async_copy(rowp.at[wid, pl.ds(g * GRP, GRP)], rowg, sem)
            pltpu.async_copy(colp.at[wid, pl.ds(g * GRP, GRP)], colg, sem)
            pltpu.async_copy(ewp.at[wid, pl.ds(g * GRP, GRP)], ewg, sem)

        def wait_group(g, bufs):
            rowg, colg, ewg, sem = bufs
            pltpu.make_async_copy(rowp.at[wid, pl.ds(g * GRP, GRP)], rowg, sem).wait()
            pltpu.make_async_copy(colp.at[wid, pl.ds(g * GRP, GRP)], colg, sem).wait()
            pltpu.make_async_copy(ewp.at[wid, pl.ds(g * GRP, GRP)], ewg, sem).wait()

        def scale(rbuf, ewg, k):
            @pl.loop(0, CHUNK, step=L)
            def _(e0):
                wv = ewg[k, pl.ds(e0, L)]
                for l in range(L):
                    w = wv[l]
                    for j in range(D // L):
                        rbuf[e0 + l, pl.ds(j * L, L)] = (
                            rbuf[e0 + l, pl.ds(j * L, L)] * w)

        HC = CHUNK // 2

        def gstart(rowg, k, rbuf, sem):
            pltpu.async_copy(hs.at[rowg.at[k, pl.ds(0, HC)]],
                             rbuf.at[pl.ds(0, HC)], sem)
            pltpu.async_copy(hs.at[rowg.at[k, pl.ds(HC, HC)]],
                             rbuf.at[pl.ds(HC, HC)], sem)

        def gwait(rowg, k, rbuf, sem):
            pltpu.make_async_copy(hs.at[rowg.at[k, pl.ds(0, HC)]],
                                  rbuf.at[pl.ds(0, HC)], sem).wait()
            pltpu.make_async_copy(hs.at[rowg.at[k, pl.ds(HC, HC)]],
                                  rbuf.at[pl.ds(HC, HC)], sem).wait()

        def half(g, cur, nxt):
            rowg, colg, ewg, _ = cur
            nrowg = nxt[0]
            for k in range(GRP):
                rb, rbsem, csem = ((rbuf0, gs0, cs0) if k % 2 == 0
                                   else (rbuf1, gs1, cs1))
                ob, obsem, ocsem = ((rbuf1, gs1, cs1) if k % 2 == 0
                                    else (rbuf0, gs0, cs0))
                gwait(rowg, k, rb, rbsem)
                if k + 1 < GRP:
                    gstart(rowg, k + 1, ob, obsem)
                else:
                    @pl.when(g + 1 < GCH)
                    def _():
                        wait_group(g + 1, nxt)
                        gstart(nrowg, 0, ob, obsem)
                scale(rb, ewg, k)
                pltpu.sync_copy(rb, acc.at[colg.at[k]], add=True)

            @pl.when(g + 2 < GCH)
            def _():
                fetch_group(g + 2, cur)

        fetch_group(0, A)
        wait_group(0, A)
        fetch_group(1, B)
        gstart(rowgA, 0, rbuf0, gs0)

        @pl.loop(0, GCH, step=2)
        def _(g):
            half(g, A, B)
            half(g + 1, B, A)

        plsc.subcore_barrier()
        off = 0
        for sz in sizes:
            pltpu.sync_copy(acc.at[pl.ds(sid * NT + off, sz)],
                            out.at[core, pl.ds(sid * NT + off, sz)])
            off += sz

    return agg_kernel


def _mm_body(x_ref, w_ref, o_ref):
    o_ref[...] = jnp.dot(x_ref[...], w_ref[...],
                         preferred_element_type=jnp.float32)


def _tc_matmul(x, W, bm=1000):
    M, K = x.shape
    Nc = W.shape[1]
    return pl.pallas_call(
        _mm_body,
        grid=(M // bm,),
        in_specs=[pl.BlockSpec((bm, K), lambda i: (i, 0)),
                  pl.BlockSpec((K, Nc), lambda i: (0, 0))],
        out_specs=pl.BlockSpec((bm, Nc), lambda i: (i, 0)),
        out_shape=jax.ShapeDtypeStruct((M, Nc), jnp.float32),
    )(x, W)


def _dis_body(dega_ref, o_ref):
    t = dega_ref[0] + dega_ref[1]
    deg = jnp.sum(t, axis=1) + 1.0
    o_ref[...] = lax.rsqrt(deg)[:, None]


def _dis_from(dega):
    NPAD = dega.shape[1]
    return pl.pallas_call(
        _dis_body,
        out_shape=jax.ShapeDtypeStruct((NPAD, 1), jnp.float32),
    )(dega)


def _rowscale_body(h_ref, dis_ref, o_ref):
    o_ref[...] = h_ref[...] * dis_ref[...]


def _rowscale(h, dis, bm=1000):
    M, D = h.shape
    return pl.pallas_call(
        _rowscale_body,
        grid=(M // bm,),
        in_specs=[pl.BlockSpec((bm, D), lambda i: (i, 0)),
                  pl.BlockSpec((bm, 1), lambda i: (i, 0))],
        out_specs=pl.BlockSpec((bm, D), lambda i: (i, 0)),
        out_shape=jax.ShapeDtypeStruct((M, D), jnp.float32),
    )(h, dis)


def _layer2_body(agg_ref, h1s_ref, dis_ref, b1_ref, o_ref):
    a = agg_ref[0] + agg_ref[1] + h1s_ref[...]
    dis = dis_ref[...]
    z = jnp.maximum(a * dis + b1_ref[...], 0.0)
    o_ref[...] = z * dis


def _layer2_in(agg, h1s, dis, b1r, bm=1000):
    M, D = h1s.shape
    return pl.pallas_call(
        _layer2_body,
        grid=(M // bm,),
        in_specs=[pl.BlockSpec((NC, bm, D), lambda i: (0, i, 0)),
                  pl.BlockSpec((bm, D), lambda i: (i, 0)),
                  pl.BlockSpec((bm, 1), lambda i: (i, 0)),
                  pl.BlockSpec((1, D), lambda i: (0, 0))],
        out_specs=pl.BlockSpec((bm, D), lambda i: (i, 0)),
        out_shape=jax.ShapeDtypeStruct((M, D), jnp.float32),
    )(agg, h1s, dis, b1r)


def _final_body(agg_ref, zs_ref, dis_ref, w2_ref, b2_ref, o_ref):
    a = agg_ref[0] + agg_ref[1] + zs_ref[...]
    h2 = jnp.dot(a, w2_ref[...], preferred_element_type=jnp.float32)
    o_ref[...] = h2 * dis_ref[...] + b2_ref[...]


def _final(agg, zs, dis, W2, b2r, bm=1000):
    M, D = zs.shape
    C = W2.shape[1]
    return pl.pallas_call(
        _final_body,
        grid=(M // bm,),
        in_specs=[pl.BlockSpec((NC, bm, D), lambda i: (0, i, 0)),
                  pl.BlockSpec((bm, D), lambda i: (i, 0)),
                  pl.BlockSpec((bm, 1), lambda i: (i, 0)),
                  pl.BlockSpec((D, C), lambda i: (0, 0)),
                  pl.BlockSpec((1, C), lambda i: (0, 0))],
        out_specs=pl.BlockSpec((bm, C), lambda i: (i, 0)),
        out_shape=jax.ShapeDtypeStruct((M, C), jnp.float32),
    )(agg, zs, dis, W2, b2r)


def kernel(x, edge_index, edge_weight, W1, b1, W2, b2):
    N, F = x.shape
    H = W1.shape[1]
    C = W2.shape[1]
    E = edge_weight.shape[0]

    CHP = -(-E // (NW * CHUNK))
    CHP = _round_up(CHP, 2 * GRP)
    EP = NW * CHUNK * CHP
    pad = EP - E

    ei = edge_index.astype(jnp.int32)
    ew = edge_weight.astype(jnp.float32)
    row, col = ei[0], ei[1]
    if pad:
        pidx = (jnp.arange(pad, dtype=jnp.int32) * 997) % N
        row = jnp.concatenate([row, pidx])
        col = jnp.concatenate([col, pidx])
        ew = jnp.concatenate([ew, jnp.zeros((pad,), jnp.float32)])
    rowp = row.reshape(NW, CHP, CHUNK)
    colp = col.reshape(NW, CHP, CHUNK)
    ewp = ew.reshape(NW, CHP, CHUNK)

    dega = _make_deg(N, CHP)(colp, ewp)
    h1 = _tc_matmul(x, W1)
    dis = _dis_from(dega)
    disN = lax.slice(dis, (0, 0), (N, 1))
    h1s = _rowscale(h1, disN)

    agg1 = _make_agg(N, H, CHP)(h1s, rowp, colp, ewp)
    zs = _layer2_in(agg1, h1s, disN, b1.reshape(1, H))

    agg2 = _make_agg(N, H, CHP)(zs, rowp, colp, ewp)
    out = _final(agg2, zs, disN, W2, b2.reshape(1, C))
    return out


# speedup vs baseline: 1.1088x; 1.1088x over previous
import dataclasses
import functools
import math

import jax
import jax.numpy as jnp
from jax import lax
from jax.experimental import pallas as pl
from jax.experimental.pallas import tpu as pltpu
from jax.experimental.pallas import tpu_sc as plsc

NC = 2
NS = 16
NW = NC * NS
L = 16
CHUNK = 128
GRP = 8

_MESH = plsc.VectorSubcoreMesh(core_axis_name="core", subcore_axis_name="subcore")

_SC_CP = pltpu.CompilerParams()
if "needs_layout_passes" in pltpu.CompilerParams.__dataclass_fields__:
    _SC_CP = dataclasses.replace(_SC_CP, needs_layout_passes=False)

_HAS_SC_TILING_FLAG = (
    "use_tc_tiling_on_sc" in pltpu.CompilerParams.__dataclass_fields__)
_SC_CP_UNTILED = (
    dataclasses.replace(_SC_CP, use_tc_tiling_on_sc=False)
    if _HAS_SC_TILING_FLAG else _SC_CP)


def _round_up(a, b):
    return -(-a // b) * b


@functools.lru_cache(maxsize=None)
def _make_deg(N, CH):
    NPAD = _round_up(N, NS * CHUNK)
    SEG = NPAD // NS

    @functools.partial(
        pl.kernel,
        out_type=jax.ShapeDtypeStruct((NC, NPAD, L), jnp.float32),
        mesh=_MESH,
        compiler_params=_SC_CP,
        scratch_types=[
            pltpu.VMEM((CH, CHUNK), jnp.int32),
            pltpu.VMEM((CH, CHUNK), jnp.float32),
            pltpu.VMEM((CHUNK, L), jnp.float32),
            pltpu.VMEM_SHARED((NPAD, L), jnp.float32),
        ],
    )
    def deg_kernel(colp, ewp, out, colv, ewv, buf, acc):
        core = lax.axis_index("core")
        sid = lax.axis_index("subcore")
        wid = core * NS + sid

        @pl.loop(0, CHUNK)
        def _(r):
            buf[r, :] = jnp.zeros((L,), jnp.float32)

        for k in range(SEG // CHUNK):
            pltpu.sync_copy(buf, acc.at[pl.ds(sid * SEG + k * CHUNK, CHUNK)])
        plsc.subcore_barrier()

        pltpu.sync_copy(colp.at[wid], colv)
        pltpu.sync_copy(ewp.at[wid], ewv)
        iota = lax.iota(jnp.int32, L)
        zcol = jnp.zeros((L,), jnp.int32)

        @pl.loop(0, CH)
        def _(i):
            for g in range(CHUNK // L):
                w = ewv[i, pl.ds(g * L, L)]
                plsc.store_scatter(buf, [g * L + iota, zcol], w)
            pltpu.sync_copy(buf, acc.at[colv.at[i]], add=True)

        plsc.subcore_barrier()
        pltpu.sync_copy(acc.at[pl.ds(sid * SEG, SEG)],
                        out.at[core, pl.ds(sid * SEG, SEG)])

    return deg_kernel


@functools.lru_cache(maxsize=None)
def _make_agg(N, D, CH, tc_tiling=True):
    GCH = CH // GRP
    NPAD = _round_up(N, NS * CHUNK)
    NT = NPAD // NS
    sizes = [CHUNK] * (NT // CHUNK) + ([NT % CHUNK] if NT % CHUNK else [])

    @functools.partial(
        pl.kernel,
        out_type=jax.ShapeDtypeStruct((NC, NPAD, D), jnp.float32),
        mesh=_MESH,
        compiler_params=_SC_CP if tc_tiling else _SC_CP_UNTILED,
        scratch_types=[
            pltpu.VMEM((GRP, CHUNK), jnp.int32),
            pltpu.VMEM((GRP, CHUNK), jnp.int32),
            pltpu.VMEM((GRP, CHUNK), jnp.int32),
            pltpu.VMEM((GRP, CHUNK), jnp.int32),
            pltpu.VMEM((GRP, CHUNK), jnp.float32),
            pltpu.VMEM((GRP, CHUNK), jnp.float32),
            pltpu.VMEM((CHUNK, D), jnp.float32),
            pltpu.VMEM((CHUNK, D), jnp.float32),
            pltpu.VMEM((1, CHUNK), jnp.int32),
            pltpu.VMEM_SHARED((NPAD, D), jnp.float32),
            pltpu.SemaphoreType.DMA,
            pltpu.SemaphoreType.DMA,
            pltpu.SemaphoreType.DMA,
            pltpu.SemaphoreType.DMA,
            pltpu.SemaphoreType.DMA,
            pltpu.SemaphoreType.DMA,
        ],
    )
    def agg_kernel(hs, rowp, colp, ewp, out,
                   rowgA, rowgB, colgA, colgB, ewgA, ewgB,
                   rbuf0, rbuf1, colx, acc, isA, isB, gs0, gs1, cs0, cs1):
        core = lax.axis_index("core")
        sid = lax.axis_index("subcore")
        wid = core * NS + sid

        @pl.loop(0, CHUNK)
        def _(r):
            for j in range(D // L):
                rbuf0[r, pl.ds(j * L, L)] = jnp.zeros((L,), jnp.float32)

        off = 0
        for sz in sizes:
            pltpu.sync_copy(rbuf0.at[pl.ds(0, sz)],
                            acc.at[pl.ds(sid * NT + off, sz)])
            off += sz
        plsc.subcore_barrier()

        A = (rowgA, colgA, ewgA, isA)
        B = (rowgB, colgB, ewgB, isB)

        def fetch_group(g, bufs):
            rowg, colg, ewg, sem = bufs
            pltpu.async_copy(rowp.at[wid, pl.ds(g * GRP, GRP)], rowg, sem)
            pltpu.async_copy(colp.at[wid, pl.ds(g * GRP, GRP)], colg, sem)
            pltpu.async_copy(ewp.at[wid, pl.ds(g * GRP, GRP)], ewg, sem)

        def wait_group(g, bufs):
            rowg, colg, ewg, sem = bufs
            pltpu.make_async_copy(rowp.at[wid, pl.ds(g * GRP, GRP)], rowg, sem).wait()
            pltpu.make_async_copy(colp.at[wid, pl.ds(g * GRP, GRP)], colg, sem).wait()
            pltpu.make_async_copy(ewp.at[wid, pl.ds(g * GRP, GRP)], ewg, sem).wait()

        def scale(rbuf, ewg, k):
            @pl.loop(0, CHUNK, step=L)
            def _(e0):
                wv = ewg[k, pl.ds(e0, L)]
                for l in range(L):
                    w = wv[l]
                    for j in range(D // L):
                        rbuf[e0 + l, pl.ds(j * L, L)] = (
                            rbuf[e0 + l, pl.ds(j * L, L)] * w)

        def gstart(rowg, k, rbuf, sem):
            pltpu.async_copy(hs.at[rowg.at[k]], rbuf, sem)

        def gwait(rowg, k, rbuf, sem):
            pltpu.make_async_copy(hs.at[rowg.at[k]], rbuf, sem).wait()

        def half(g, cur, nxt):
            rowg, colg, ewg, _ = cur
            nrowg = nxt[0]
            for k in range(GRP):
                rb, rbsem, csem = ((rbuf0, gs0, cs0) if k % 2 == 0
                                   else (rbuf1, gs1, cs1))
                ob, obsem, ocsem = ((rbuf1, gs1, cs1) if k % 2 == 0
                                    else (rbuf0, gs0, cs0))
                gwait(rowg, k, rb, rbsem)
                if k + 1 < GRP:
                    gstart(rowg, k + 1, ob, obsem)
                else:
                    @pl.when(g + 1 < GCH)
                    def _():
                        wait_group(g + 1, nxt)
                        gstart(nrowg, 0, ob, obsem)
                scale(rb, ewg, k)
                pltpu.sync_copy(rb, acc.at[colg.at[k]], add=True)

            @pl.when(g + 2 < GCH)
            def _():
                fetch_group(g + 2, cur)

        fetch_group(0, A)
        wait_group(0, A)
        fetch_group(1, B)
        gstart(rowgA, 0, rbuf0, gs0)

        @pl.loop(0, GCH, step=2)
        def _(g):
            half(g, A, B)
            half(g + 1, B, A)

        plsc.subcore_barrier()
        off = 0
        for sz in sizes:
            pltpu.sync_copy(acc.at[pl.ds(sid * NT + off, sz)],
                            out.at[core, pl.ds(sid * NT + off, sz)])
            off += sz

    return agg_kernel


def _mm_body(x_ref, w_ref, o_ref):
    o_ref[...] = jnp.dot(x_ref[...], w_ref[...],
                         preferred_element_type=jnp.float32)


def _tc_matmul(x, W, bm=1000):
    M, K = x.shape
    Nc = W.shape[1]
    return pl.pallas_call(
        _mm_body,
        grid=(M // bm,),
        in_specs=[pl.BlockSpec((bm, K), lambda i: (i, 0)),
                  pl.BlockSpec((K, Nc), lambda i: (0, 0))],
        out_specs=pl.BlockSpec((bm, Nc), lambda i: (i, 0)),
        out_shape=jax.ShapeDtypeStruct((M, Nc), jnp.float32),
    )(x, W)


def _dis_body(dega_ref, o_ref):
    t = dega_ref[0] + dega_ref[1]
    deg = jnp.sum(t, axis=1) + 1.0
    o_ref[...] = lax.rsqrt(deg)[:, None]


def _dis_from(dega):
    NPAD = dega.shape[1]
    return pl.pallas_call(
        _dis_body,
        out_shape=jax.ShapeDtypeStruct((NPAD, 1), jnp.float32),
    )(dega)


def _rowscale_body(h_ref, dis_ref, o_ref):
    o_ref[...] = h_ref[...] * dis_ref[...]


def _rowscale(h, dis, bm=1000):
    M, D = h.shape
    return pl.pallas_call(
        _rowscale_body,
        grid=(M // bm,),
        in_specs=[pl.BlockSpec((bm, D), lambda i: (i, 0)),
                  pl.BlockSpec((bm, 1), lambda i: (i, 0))],
        out_specs=pl.BlockSpec((bm, D), lambda i: (i, 0)),
        out_shape=jax.ShapeDtypeStruct((M, D), jnp.float32),
    )(h, dis)


def _layer2_body(agg_ref, h1s_ref, dis_ref, b1_ref, w2_ref, o_ref):
    a = agg_ref[0] + agg_ref[1] + h1s_ref[...]
    dis = dis_ref[...]
    z = jnp.maximum(a * dis + b1_ref[...], 0.0)
    o_ref[...] = jnp.dot(z, w2_ref[...],
                         preferred_element_type=jnp.float32) * dis


def _layer2_in(agg, h1s, dis, b1r, W2p, bm=1000):
    M, D = h1s.shape
    DP = W2p.shape[1]
    return pl.pallas_call(
        _layer2_body,
        grid=(M // bm,),
        in_specs=[pl.BlockSpec((NC, bm, D), lambda i: (0, i, 0)),
                  pl.BlockSpec((bm, D), lambda i: (i, 0)),
                  pl.BlockSpec((bm, 1), lambda i: (i, 0)),
                  pl.BlockSpec((1, D), lambda i: (0, 0)),
                  pl.BlockSpec((D, DP), lambda i: (0, 0))],
        out_specs=pl.BlockSpec((bm, DP), lambda i: (i, 0)),
        out_shape=jax.ShapeDtypeStruct((M, DP), jnp.float32),
    )(agg, h1s, dis, b1r, W2p)


def _final_body(C, agg_ref, h2s_ref, dis_ref, b2_ref, o_ref):
    a = agg_ref[0] + agg_ref[1] + h2s_ref[...]
    o_ref[...] = (a * dis_ref[...])[:, :C] + b2_ref[...]


def _final(agg, h2s, dis, b2r, C, bm=1000):
    M, DP = h2s.shape
    return pl.pallas_call(
        functools.partial(_final_body, C),
        grid=(M // bm,),
        in_specs=[pl.BlockSpec((NC, bm, DP), lambda i: (0, i, 0)),
                  pl.BlockSpec((bm, DP), lambda i: (i, 0)),
                  pl.BlockSpec((bm, 1), lambda i: (i, 0)),
                  pl.BlockSpec((1, C), lambda i: (0, 0))],
        out_specs=pl.BlockSpec((bm, C), lambda i: (i, 0)),
        out_shape=jax.ShapeDtypeStruct((M, C), jnp.float32),
    )(agg, h2s, dis, b2r)


def kernel(x, edge_index, edge_weight, W1, b1, W2, b2):
    N, F = x.shape
    H = W1.shape[1]
    C = W2.shape[1]
    E = edge_weight.shape[0]

    CHP = -(-E // (NW * CHUNK))
    CHP = _round_up(CHP, 2 * GRP)
    EP = NW * CHUNK * CHP
    pad = EP - E

    ei = edge_index.astype(jnp.int32)
    ew = edge_weight.astype(jnp.float32)
    row, col = ei[0], ei[1]
    if pad:
        pidx = (jnp.arange(pad, dtype=jnp.int32) * 997) % N
        row = jnp.concatenate([row, pidx])
        col = jnp.concatenate([col, pidx])
        ew = jnp.concatenate([ew, jnp.zeros((pad,), jnp.float32)])
    rowp = row.reshape(NW, CHP, CHUNK)
    colp = col.reshape(NW, CHP, CHUNK)
    ewp = ew.reshape(NW, CHP, CHUNK)

    dega = _make_deg(N, CHP)(colp, ewp)
    h1 = _tc_matmul(x, W1)
    dis = _dis_from(dega)
    disN = lax.slice(dis, (0, 0), (N, 1))
    h1s = _rowscale(h1, disN)

    agg1 = _make_agg(N, H, CHP)(h1s, rowp, colp, ewp)

    DP = _round_up(C, L)
    W2p = jnp.pad(W2, ((0, 0), (0, DP - C)))
    h2s = _layer2_in(agg1, h1s, disN, b1.reshape(1, H), W2p)

    agg2 = _make_agg(N, DP, CHP, tc_tiling=False)(h2s, rowp, colp, ewp)
    out = _final(agg2, h2s, disN, b2.reshape(1, C), C)
    return out
